# baseline (speedup 1.0000x reference)
import functools

import jax
import jax.numpy as jnp
from jax import lax
from jax.experimental import pallas as pl
from jax.experimental.pallas import tpu as pltpu
from jax.experimental.pallas import tpu_sc as plsc

NC, NS, L = 2, 16, 16
NW = NC * NS

_LN2 = 0.6931471805599453
_SQRT2 = 1.4142135623730951


def _ln(x):
    bits = plsc.bitcast(x, jnp.int32)
    e = ((bits >> 23) & 0xFF) - 127
    m = plsc.bitcast((bits & 0x7FFFFF) | (127 << 23), jnp.float32)
    ef = e.astype(jnp.float32)
    big = m > _SQRT2
    m = jnp.where(big, m * 0.5, m)
    ef = jnp.where(big, ef + 1.0, ef)
    z = (m - 1.0) / (m + 1.0)
    z2 = z * z
    p = 1.0 / 11.0
    p = p * z2 + 1.0 / 9.0
    p = p * z2 + 1.0 / 7.0
    p = p * z2 + 1.0 / 5.0
    p = p * z2 + 1.0 / 3.0
    p = p * z2 + 1.0
    return ef * _LN2 + 2.0 * z * p


def _make_sc_partials(B, D):
    bpw = B // NW
    nchunks = bpw // L
    mesh = plsc.VectorSubcoreMesh(
        core_axis_name="c", subcore_axis_name="s", num_cores=NC, num_subcores=NS
    )

    @functools.partial(
        pl.kernel,
        out_type=jax.ShapeDtypeStruct((NW, L), jnp.float32),
        mesh=mesh,
        scratch_types=[
            pltpu.VMEM((bpw,), jnp.int32),
            pltpu.VMEM((bpw,), jnp.int32),
            pltpu.VMEM((bpw,), jnp.int32),
            pltpu.VMEM((bpw,), jnp.int32),
            pltpu.VMEM((bpw, 2 * D), jnp.float32),
            pltpu.VMEM((bpw, 2 * D), jnp.float32),
            pltpu.VMEM((bpw,), jnp.float32),
            pltpu.VMEM((bpw,), jnp.float32),
            pltpu.VMEM((bpw,), jnp.float32),
            pltpu.VMEM((bpw,), jnp.float32),
            pltpu.VMEM((1, L), jnp.float32),
            pltpu.SemaphoreType.DMA,
        ],
        compiler_params=pltpu.CompilerParams(
            needs_layout_passes=False, use_tc_tiling_on_sc=False
        ),
    )
    def sc_partials(i_hbm, j_hbm, co_hbm, wt_hbm, v_hbm, w_hbm, bv_hbm, bw_hbm,
                    out_hbm, idx_i, idx_j, gi, gj, rows_v, rows_w, bi, bj,
                    co_v, wt_v, part, sem):
        cid = lax.axis_index("c")
        sid = lax.axis_index("s")
        wid = sid * NC + cid
        base = wid * bpw

        pltpu.sync_copy(i_hbm.at[pl.ds(base, bpw)], idx_i)
        pltpu.sync_copy(j_hbm.at[pl.ds(base, bpw)], idx_j)
        for ch in range(nchunks):
            sl = pl.ds(ch * L, L)
            gi[sl] = idx_i[sl] >> 1
            gj[sl] = idx_j[sl] >> 1
        pltpu.sync_copy(co_hbm.at[pl.ds(base, bpw)], co_v)
        pltpu.sync_copy(wt_hbm.at[pl.ds(base, bpw)], wt_v)

        cp1 = pltpu.async_copy(v_hbm.at[gi], rows_v, sem)
        cp2 = pltpu.async_copy(w_hbm.at[gj], rows_w, sem)
        cp3 = pltpu.async_copy(bv_hbm.at[idx_i], bi, sem)
        cp4 = pltpu.async_copy(bw_hbm.at[idx_j], bj, sem)
        cp1.wait()
        cp2.wait()
        cp3.wait()
        cp4.wait()

        iota = lax.broadcasted_iota(jnp.int32, (L,), 0)
        zerosf = jnp.zeros((L,), jnp.float32)
        acc_s1 = zerosf
        acc_s2 = zerosf
        acc_s3 = zerosf
        acc_t1 = zerosf
        acc_t2 = zerosf
        for ch in range(nchunks):
            sl = pl.ds(ch * L, L)
            ridx = iota + ch * L
            half_i = (idx_i[sl] & 1) * D
            half_j = (idx_j[sl] & 1) * D

            def dot_body(d, acc):
                va = plsc.load_gather(rows_v, [ridx, half_i + d])
                vb = plsc.load_gather(rows_w, [ridx, half_j + d])
                return acc + va * vb

            sim = lax.fori_loop(0, D, dot_body, zerosf)
            a = sim - _ln(co_v[sl])
            wt_c = wt_v[sl]
            wa = wt_c * a
            acc_s1 = acc_s1 + wa * a
            acc_s2 = acc_s2 + wa
            acc_s3 = acc_s3 + wt_c
            t = bi[sl] + bj[sl]
            acc_t1 = acc_t1 + t
            acc_t2 = acc_t2 + t * t

        s1 = jnp.sum(acc_s1)
        s2 = jnp.sum(acc_s2)
        s3 = jnp.sum(acc_s3)
        t1 = jnp.sum(acc_t1)
        t2 = jnp.sum(acc_t2)
        outv = jnp.where(iota == 0, s1, 0.0)
        outv = outv + jnp.where(iota == 1, s2, 0.0)
        outv = outv + jnp.where(iota == 2, s3, 0.0)
        outv = outv + jnp.where(iota == 3, t1, 0.0)
        outv = outv + jnp.where(iota == 4, t2, 0.0)
        part[0, :] = outv
        pltpu.sync_copy(part, out_hbm.at[pl.ds(wid, 1)])

    return sc_partials


def kernel(i, j, co_occur, weight, V, W, BV, BW):
    B = i.shape[0]
    VOCAB, D = V.shape
    VP = jnp.reshape(V, (VOCAB // 2, 2 * D))
    WP = jnp.reshape(W, (VOCAB // 2, 2 * D))
    p = _make_sc_partials(B, D)(
        i, j, co_occur, weight, VP, WP,
        jnp.reshape(BV, (-1,)), jnp.reshape(BW, (-1,))
    )
    s1 = jnp.sum(p[:, 0])
    s2 = jnp.sum(p[:, 1])
    s3 = jnp.sum(p[:, 2])
    t1 = jnp.sum(p[:, 3])
    t2 = jnp.sum(p[:, 4])
    return 0.5 * (B * s1 + 2.0 * t1 * s2 + t2 * s3)


# speedup vs baseline: 1.3809x; 1.3809x over previous
import functools

import jax
import jax.numpy as jnp
from jax import lax
from jax.experimental import pallas as pl
from jax.experimental.pallas import tpu as pltpu
from jax.experimental.pallas import tpu_sc as plsc

NC, NS, L = 2, 16, 16
NW = NC * NS
RB = 32

_LN2 = 0.6931471805599453
_SQRT2 = 1.4142135623730951


def _ln(x):
    bits = plsc.bitcast(x, jnp.int32)
    e = ((bits >> 23) & 0xFF) - 127
    m = plsc.bitcast((bits & 0x7FFFFF) | (127 << 23), jnp.float32)
    ef = e.astype(jnp.float32)
    big = m > _SQRT2
    m = jnp.where(big, m * 0.5, m)
    ef = jnp.where(big, ef + 1.0, ef)
    z = (m - 1.0) / (m + 1.0)
    z2 = z * z
    p = 1.0 / 11.0
    p = p * z2 + 1.0 / 9.0
    p = p * z2 + 1.0 / 7.0
    p = p * z2 + 1.0 / 5.0
    p = p * z2 + 1.0 / 3.0
    p = p * z2 + 1.0
    return ef * _LN2 + 2.0 * z * p


def _make_sc_partials(B, D):
    bpw = B // NW
    nrounds = bpw // RB
    mesh = plsc.VectorSubcoreMesh(
        core_axis_name="c", subcore_axis_name="s", num_cores=NC, num_subcores=NS
    )

    @functools.partial(
        pl.kernel,
        out_type=jax.ShapeDtypeStruct((NW, L), jnp.float32),
        mesh=mesh,
        scratch_types=[
            pltpu.VMEM((bpw,), jnp.int32),
            pltpu.VMEM((bpw,), jnp.int32),
            pltpu.VMEM((RB, 8, D), jnp.float32),
            pltpu.VMEM((RB, 8, D), jnp.float32),
            pltpu.VMEM((bpw,), jnp.float32),
            pltpu.VMEM((bpw,), jnp.float32),
            pltpu.VMEM((bpw,), jnp.float32),
            pltpu.VMEM((bpw,), jnp.float32),
            pltpu.VMEM((1, L), jnp.float32),
            pltpu.SemaphoreType.DMA,
            pltpu.SemaphoreType.DMA,
        ],
        compiler_params=pltpu.CompilerParams(
            needs_layout_passes=False, use_tc_tiling_on_sc=True
        ),
    )
    def sc_partials(i_hbm, j_hbm, co_hbm, wt_hbm, v_hbm, w_hbm, bv_hbm, bw_hbm,
                    out_hbm, idx_i, idx_j, slabs_v, slabs_w, bi, bj,
                    co_v, wt_v, part, sem, semb):
        cid = lax.axis_index("c")
        sid = lax.axis_index("s")
        wid = sid * NC + cid
        base = wid * bpw

        pltpu.sync_copy(i_hbm.at[pl.ds(base, bpw)], idx_i)
        pltpu.sync_copy(j_hbm.at[pl.ds(base, bpw)], idx_j)
        pltpu.sync_copy(co_hbm.at[pl.ds(base, bpw)], co_v)
        pltpu.sync_copy(wt_hbm.at[pl.ds(base, bpw)], wt_v)
        cp3 = pltpu.async_copy(bv_hbm.at[idx_i], bi, semb)
        cp4 = pltpu.async_copy(bw_hbm.at[idx_j], bj, semb)

        iota = lax.broadcasted_iota(jnp.int32, (L,), 0)
        zerosf = jnp.zeros((L,), jnp.float32)
        acc_s1 = zerosf
        acc_s2 = zerosf
        acc_s3 = zerosf
        acc_t1 = zerosf
        acc_t2 = zerosf
        for r in range(nrounds):
            vv = [idx_i[pl.ds(r * RB + c * L, L)] for c in range(RB // L)]
            wv = [idx_j[pl.ds(r * RB + c * L, L)] for c in range(RB // L)]
            cps = []
            for c in range(RB // L):
                for e in range(L):
                    s = c * L + e
                    gv = pl.multiple_of(vv[c][e] & ~7, 8)
                    gw = pl.multiple_of(wv[c][e] & ~7, 8)
                    cps.append(pltpu.async_copy(
                        v_hbm.at[pl.ds(gv, 8), :], slabs_v.at[s], sem))
                    cps.append(pltpu.async_copy(
                        w_hbm.at[pl.ds(gw, 8), :], slabs_w.at[s], sem))
            for cp in cps:
                cp.wait()
            for c in range(RB // L):
                sl = pl.ds(r * RB + c * L, L)
                slab_idx = iota + c * L
                ri = vv[c] & 7
                rj = wv[c] & 7

                def dot_body(d, acc):
                    va = plsc.load_gather(slabs_v, [slab_idx, ri, iota * 0 + d])
                    vb = plsc.load_gather(slabs_w, [slab_idx, rj, iota * 0 + d])
                    return acc + va * vb

                sim = lax.fori_loop(0, D, dot_body, zerosf)
                a = sim - _ln(co_v[sl])
                wt_c = wt_v[sl]
                wa = wt_c * a
                acc_s1 = acc_s1 + wa * a
                acc_s2 = acc_s2 + wa
                acc_s3 = acc_s3 + wt_c

        cp3.wait()
        cp4.wait()
        for ch in range(bpw // L):
            sl = pl.ds(ch * L, L)
            t = bi[sl] + bj[sl]
            acc_t1 = acc_t1 + t
            acc_t2 = acc_t2 + t * t

        s1 = jnp.sum(acc_s1)
        s2 = jnp.sum(acc_s2)
        s3 = jnp.sum(acc_s3)
        t1 = jnp.sum(acc_t1)
        t2 = jnp.sum(acc_t2)
        outv = jnp.where(iota == 0, s1, 0.0)
        outv = outv + jnp.where(iota == 1, s2, 0.0)
        outv = outv + jnp.where(iota == 2, s3, 0.0)
        outv = outv + jnp.where(iota == 3, t1, 0.0)
        outv = outv + jnp.where(iota == 4, t2, 0.0)
        part[0, :] = outv
        pltpu.sync_copy(part, out_hbm.at[pl.ds(wid, 1)])

    return sc_partials


def kernel(i, j, co_occur, weight, V, W, BV, BW):
    B = i.shape[0]
    D = V.shape[1]
    p = _make_sc_partials(B, D)(
        i, j, co_occur, weight, V, W,
        jnp.reshape(BV, (-1,)), jnp.reshape(BW, (-1,))
    )
    s1 = jnp.sum(p[:, 0])
    s2 = jnp.sum(p[:, 1])
    s3 = jnp.sum(p[:, 2])
    t1 = jnp.sum(p[:, 3])
    t2 = jnp.sum(p[:, 4])
    return 0.5 * (B * s1 + 2.0 * t1 * s2 + t2 * s3)


# speedup vs baseline: 2.0202x; 1.4630x over previous
import functools

import jax
import jax.numpy as jnp
from jax import lax
from jax.experimental import pallas as pl
from jax.experimental.pallas import tpu as pltpu
from jax.experimental.pallas import tpu_sc as plsc

NC, NS, L = 2, 16, 16
NW = NC * NS
RB = 32

_LN2 = 0.6931471805599453
_SQRT2 = 1.4142135623730951


def _ln(x):
    bits = plsc.bitcast(x, jnp.int32)
    e = ((bits >> 23) & 0xFF) - 127
    m = plsc.bitcast((bits & 0x7FFFFF) | (127 << 23), jnp.float32)
    ef = e.astype(jnp.float32)
    big = m > _SQRT2
    m = jnp.where(big, m * 0.5, m)
    ef = jnp.where(big, ef + 1.0, ef)
    z = (m - 1.0) / (m + 1.0)
    z2 = z * z
    p = 1.0 / 11.0
    p = p * z2 + 1.0 / 9.0
    p = p * z2 + 1.0 / 7.0
    p = p * z2 + 1.0 / 5.0
    p = p * z2 + 1.0 / 3.0
    p = p * z2 + 1.0
    return ef * _LN2 + 2.0 * z * p


def _make_sc_partials(B, D):
    bpw = B // NW
    nrounds = bpw // RB
    mesh = plsc.VectorSubcoreMesh(
        core_axis_name="c", subcore_axis_name="s", num_cores=NC, num_subcores=NS
    )

    @functools.partial(
        pl.kernel,
        out_type=jax.ShapeDtypeStruct((NW, L), jnp.float32),
        mesh=mesh,
        scratch_types=[
            pltpu.VMEM((bpw,), jnp.int32),
            pltpu.VMEM((bpw,), jnp.int32),
            pltpu.VMEM((RB, 8, D), jnp.float32),
            pltpu.VMEM((RB, 8, D), jnp.float32),
            pltpu.VMEM((bpw,), jnp.float32),
            pltpu.VMEM((bpw,), jnp.float32),
            pltpu.VMEM((bpw,), jnp.float32),
            pltpu.VMEM((bpw,), jnp.float32),
            pltpu.VMEM((1, L), jnp.float32),
            pltpu.SemaphoreType.DMA,
            pltpu.SemaphoreType.DMA,
        ],
        compiler_params=pltpu.CompilerParams(
            needs_layout_passes=False, use_tc_tiling_on_sc=True
        ),
    )
    def sc_partials(i_hbm, j_hbm, co_hbm, wt_hbm, v3_hbm, w_hbm, bv_hbm, bw_hbm,
                    out_hbm, idx_i, idx_j, slabs_v, slabs_w, bi, bj,
                    co_v, wt_v, part, sem, semb):
        cid = lax.axis_index("c")
        sid = lax.axis_index("s")
        wid = sid * NC + cid
        base = wid * bpw

        pltpu.sync_copy(i_hbm.at[pl.ds(base, bpw)], idx_i)
        pltpu.sync_copy(j_hbm.at[pl.ds(base, bpw)], idx_j)
        pltpu.sync_copy(co_hbm.at[pl.ds(base, bpw)], co_v)
        pltpu.sync_copy(wt_hbm.at[pl.ds(base, bpw)], wt_v)
        cp3 = pltpu.async_copy(bv_hbm.at[idx_i], bi, semb)
        cp4 = pltpu.async_copy(bw_hbm.at[idx_j], bj, semb)

        iota = lax.broadcasted_iota(jnp.int32, (L,), 0)
        zerosf = jnp.zeros((L,), jnp.float32)
        acc_s1 = zerosf
        acc_s2 = zerosf
        acc_s3 = zerosf
        acc_t1 = zerosf
        acc_t2 = zerosf
        for r in range(nrounds):
            vv = [idx_i[pl.ds(r * RB + c * L, L)] for c in range(RB // L)]
            wv = [idx_j[pl.ds(r * RB + c * L, L)] for c in range(RB // L)]
            cps = []
            for c in range(RB // L):
                for e in range(L):
                    s = c * L + e
                    gv = vv[c][e] >> 3
                    gw = pl.multiple_of(wv[c][e] & ~7, 8)
                    cps.append(pltpu.async_copy(
                        v3_hbm.at[gv], slabs_v.at[s], sem))
                    cps.append(pltpu.async_copy(
                        w_hbm.at[pl.ds(gw, 8), :], slabs_w.at[s], sem))
            for cp in cps:
                cp.wait()
            for c in range(RB // L):
                sl = pl.ds(r * RB + c * L, L)
                slab_idx = iota + c * L
                ri = vv[c] & 7
                rj = wv[c] & 7

                def dot_body(d, acc):
                    va = plsc.load_gather(slabs_v, [slab_idx, ri, iota * 0 + d])
                    vb = plsc.load_gather(slabs_w, [slab_idx, rj, iota * 0 + d])
                    return acc + va * vb

                sim = lax.fori_loop(0, D, dot_body, zerosf)
                a = sim - _ln(co_v[sl])
                wt_c = wt_v[sl]
                wa = wt_c * a
                acc_s1 = acc_s1 + wa * a
                acc_s2 = acc_s2 + wa
                acc_s3 = acc_s3 + wt_c

        cp3.wait()
        cp4.wait()
        for ch in range(bpw // L):
            sl = pl.ds(ch * L, L)
            t = bi[sl] + bj[sl]
            acc_t1 = acc_t1 + t
            acc_t2 = acc_t2 + t * t

        s1 = jnp.sum(acc_s1)
        s2 = jnp.sum(acc_s2)
        s3 = jnp.sum(acc_s3)
        t1 = jnp.sum(acc_t1)
        t2 = jnp.sum(acc_t2)
        outv = jnp.where(iota == 0, s1, 0.0)
        outv = outv + jnp.where(iota == 1, s2, 0.0)
        outv = outv + jnp.where(iota == 2, s3, 0.0)
        outv = outv + jnp.where(iota == 3, t1, 0.0)
        outv = outv + jnp.where(iota == 4, t2, 0.0)
        part[0, :] = outv
        pltpu.sync_copy(part, out_hbm.at[pl.ds(wid, 1)])

    return sc_partials


def kernel(i, j, co_occur, weight, V, W, BV, BW):
    B = i.shape[0]
    D = V.shape[1]
    VOCAB = V.shape[0]
    V3 = jnp.reshape(V, (VOCAB // 8, 8, D))
    p = _make_sc_partials(B, D)(
        i, j, co_occur, weight, V3, W,
        jnp.reshape(BV, (-1,)), jnp.reshape(BW, (-1,))
    )
    s1 = jnp.sum(p[:, 0])
    s2 = jnp.sum(p[:, 1])
    s3 = jnp.sum(p[:, 2])
    t1 = jnp.sum(p[:, 3])
    t2 = jnp.sum(p[:, 4])
    return 0.5 * (B * s1 + 2.0 * t1 * s2 + t2 * s3)
